# baseline (speedup 1.0000x reference)
import functools

import jax
import jax.numpy as jnp
from jax.experimental import pallas as pl
from jax.experimental.pallas import tpu as pltpu
from jax.experimental.pallas import tpu_sc as plsc

_B, _H, _W, _K, _C = 1, 384, 384, 4, 96
_ROWS = _H * _W
_R = 128
_LANES = 16


def kernel(texels, pix_to_face):
    tex2 = texels.reshape(_ROWS, _K * _C)
    p1 = pix_to_face.reshape(_ROWS * _K)
    mesh = plsc.VectorSubcoreMesh(core_axis_name="c", subcore_axis_name="s")

    @functools.partial(
        pl.kernel,
        out_type=jax.ShapeDtypeStruct((_ROWS, _C), jnp.float32),
        mesh=mesh,
        compiler_params=pltpu.CompilerParams(use_tc_tiling_on_sc=False),
    )
    def masked_copy(tex_hbm, p_hbm, out_hbm):
        ones = jnp.ones((_LANES,), jnp.float32)
        zeros = jnp.zeros((_LANES,), jnp.float32)

        def body(tex_vmem, p_vmem, out_vmem):
            @pl.loop(0, _R // 4)
            def _(g):
                p16 = p_vmem[pl.ds(_LANES * g, _LANES)]
                svec = jnp.where(p16 >= 0, ones, zeros)
                for j in range(4):
                    r = 4 * g + j
                    vs = jax.lax.broadcast_in_dim(svec[4 * j], (_LANES,), ())
                    for c in range(0, _C, _LANES):
                        v = tex_vmem[r, pl.ds(c, _LANES)]
                        out_vmem[r, pl.ds(c, _LANES)] = v * vs

        pltpu.emit_pipeline(
            body,
            grid=(_ROWS // _R,),
            in_specs=[
                pl.BlockSpec((_R, _C), lambda i: (i, 0)),
                pl.BlockSpec((_R * _K,), lambda i: (i,)),
            ],
            out_specs=[pl.BlockSpec((_R, _C), lambda i: (i, 0))],
            core_axis_name=("c", "s"),
            dimension_semantics=(pltpu.PARALLEL,),
        )(tex_hbm, p_hbm, out_hbm)

    out = masked_copy(tex2, p1)
    return out.reshape(_B, _H, _W, _C)
